# baseline (speedup 1.0000x reference)
import jax
import jax.numpy as jnp
from jax.experimental import pallas as pl
from jax.experimental.pallas import tpu as pltpu


def _make_stage1_body(G, Cin, Cout, KH, KW, PH, PW, chunks):
    OHu = 2 * PH

    def body(x_ref, wc6_ref, wc4_ref, b6_ref, b4_ref, o_ref):
        W = x_ref.shape[2] // Cin
        for (pw0, ncols) in chunks:
            base = 2 * pw0
            width = KW + (ncols - 1)
            w_ref = wc6_ref if ncols == 6 else wc4_ref
            b_ref = b6_ref if ncols == 6 else b4_ref
            bands = [x_ref[:, :, c * W + base:c * W + base + width]
                     for c in range(Cin)]
            pieces = [bands[c][i:i + OHu] for i in range(KH) for c in range(Cin)]
            lhs = jnp.concatenate(pieces, axis=2).reshape(OHu * G,
                                                          KH * Cin * width)
            y = jnp.dot(lhs, w_ref[...], preferred_element_type=jnp.float32)
            y = jnp.maximum(y + b_ref[...], 0.0)
            y3 = y.reshape(OHu, G, ncols * Cout)
            for s in range(ncols // 2):
                m = jnp.maximum(y3[:, :, (2 * s) * Cout:(2 * s + 1) * Cout],
                                y3[:, :, (2 * s + 1) * Cout:(2 * s + 2) * Cout])
                v = jnp.max(m.reshape(PH, 2, G, Cout), axis=1)
                pw = pw0 + s
                o_ref[:, :, pw * Cout:(pw + 1) * Cout] = v.astype(jnp.bfloat16)
    return body


def _stage12_fused(x2d, w1, b1, w2, b2):
    H, N, WC = x2d.shape
    Cin, Cout, KH, KW = 3, 128, 7, 7
    W = WC // Cin
    OH, OW = H - KH + 1, W - KW + 1
    PH, PW = OH // 2, OW // 2

    w1r = jnp.transpose(w1.reshape(KH, KW, Cin, Cout), (0, 2, 1, 3))

    def widen(ncols):
        cols = [jnp.pad(w1r, ((0, 0), (0, 0), (s, ncols - 1 - s), (0, 0)))
                for s in range(ncols)]
        return jnp.stack(cols, axis=3).reshape(KH * Cin * (KW + ncols - 1),
                                               ncols * Cout)

    w1c6 = widen(6)
    w1c4 = widen(4)
    b6 = jnp.concatenate([b1] * 6, axis=1)
    b4 = jnp.concatenate([b1] * 4, axis=1)

    chunks = []
    pw = 0
    while pw < PW:
        n = 3 if PW - pw >= 3 else PW - pw
        chunks.append((pw, 2 * n))
        pw += n

    G = 16 if N % 16 == 0 else (8 if N % 8 == 0 else 1)
    s1body = _make_stage1_body(G, Cin, Cout, KH, KW, PH, PW, chunks)

    C2in, C2out, K2H, K2W = Cout, 256, 5, 5
    O2H, O2W = PH - K2H + 1, PW - K2W + 1
    P2H, P2W = O2H // 2, O2W // 2
    s2body = _make_stage2_body(G, C2in, C2out, K2H, K2W, P2H, P2W)

    def body(x_ref, wc6_ref, wc4_ref, b6_ref, b4_ref, w2_ref, b2_ref,
             o_ref, h1_ref):
        s1body(x_ref, wc6_ref, wc4_ref, b6_ref, b4_ref, h1_ref)
        s2body(h1_ref, w2_ref, b2_ref, o_ref)

    return pl.pallas_call(
        body,
        out_shape=jax.ShapeDtypeStruct((P2H, N, P2W * C2out), jnp.bfloat16),
        grid=(N // G,),
        in_specs=[
            pl.BlockSpec((H, G, WC), lambda n: (0, n, 0)),
            pl.BlockSpec(w1c6.shape, lambda n: (0, 0)),
            pl.BlockSpec(w1c4.shape, lambda n: (0, 0)),
            pl.BlockSpec(b6.shape, lambda n: (0, 0)),
            pl.BlockSpec(b4.shape, lambda n: (0, 0)),
            pl.BlockSpec(w2.shape, lambda n: (0, 0)),
            pl.BlockSpec(b2.shape, lambda n: (0, 0)),
        ],
        out_specs=pl.BlockSpec((P2H, G, P2W * C2out), lambda n: (0, n, 0)),
        scratch_shapes=[pltpu.VMEM((PH, G, PW * Cout), jnp.bfloat16)],
        compiler_params=pltpu.CompilerParams(dimension_semantics=("parallel",)),
    )(x2d, w1c6, w1c4, b6, b4, w2, b2)


def _make_stage2_body(G, Cin, Cout, KH, KW, PH, PW):
    OHu = 2 * PH
    KWC = KW * Cin

    def conv_col(h_ref, w_ref, b_ref, ow):
        pieces = [h_ref[i:i + OHu, :, ow * Cin:ow * Cin + KWC] for i in range(KH)]
        lhs = jnp.concatenate(pieces, axis=2).reshape(OHu * G, KH * KWC)
        acc = jnp.dot(lhs, w_ref[...], preferred_element_type=jnp.float32)
        return jnp.maximum(acc + b_ref[...], 0.0)

    def body(h_ref, w_ref, b_ref, o_ref):
        for pw in range(PW):
            cmax = jnp.maximum(conv_col(h_ref, w_ref, b_ref, 2 * pw),
                               conv_col(h_ref, w_ref, b_ref, 2 * pw + 1))
            pooled = jnp.max(cmax.reshape(PH, 2, G, Cout), axis=1)
            o_ref[:, :, pw * Cout:(pw + 1) * Cout] = pooled.astype(jnp.bfloat16)
    return body


def _make_linear_body(PH, FW):
    def body(f_ref, w_ref, b_ref, o_ref):
        y = b_ref[...]
        for ph in range(PH):
            wl = w_ref[ph * FW:(ph + 1) * FW, :].astype(jnp.bfloat16)
            y = y + jnp.dot(f_ref[ph], wl, preferred_element_type=jnp.float32)
        o_ref[...] = y
    return body


def _linear(feats, w_lin, b_lin):
    PH, N, FW = feats.shape
    out = w_lin.shape[1]
    G = N // 2 if N % 2 == 0 else N
    return pl.pallas_call(
        _make_linear_body(PH, FW),
        out_shape=jax.ShapeDtypeStruct((N, out), jnp.float32),
        grid=(N // G,),
        in_specs=[
            pl.BlockSpec((PH, G, FW), lambda n: (0, n, 0)),
            pl.BlockSpec(w_lin.shape, lambda n: (0, 0)),
            pl.BlockSpec(b_lin.shape, lambda n: (0, 0)),
        ],
        out_specs=pl.BlockSpec((G, out), lambda n: (n, 0)),
        compiler_params=pltpu.CompilerParams(dimension_semantics=("parallel",)),
    )(feats, w_lin, b_lin)


@jax.jit
def _forward(x, w1, b1, w2, b2, w_lin, b_lin):
    N, C, H, W = x.shape
    x2d = jnp.transpose(x, (2, 0, 1, 3)).reshape(H, N, C * W).astype(jnp.bfloat16)
    feats = _stage12_fused(x2d, w1, b1, w2, b2)
    return _linear(feats, w_lin, b_lin)


def kernel(x, w1, b1, w2, b2, w_lin, b_lin):
    return _forward(x, w1, b1, w2, b2, w_lin, b_lin)


# speedup vs baseline: 1.2543x; 1.2543x over previous
import jax
import jax.numpy as jnp
from jax.experimental import pallas as pl
from jax.experimental.pallas import tpu as pltpu


def _make_stage1_body(G, Cin, Cout, KH, KW, PH, PW, chunks):
    OHu = 2 * PH

    def body(x_ref, wc6_ref, wc4_ref, b6_ref, b4_ref, o_ref):
        for (pw0, ncols) in chunks:
            base = 2 * pw0 * Cin
            width = KW * Cin + (ncols - 1) * Cin
            w_ref = wc6_ref if ncols == 6 else wc4_ref
            b_ref = b6_ref if ncols == 6 else b4_ref
            band = x_ref[:, :, base:base + width]
            pieces = [band[i:i + OHu] for i in range(KH)]
            lhs = jnp.concatenate(pieces, axis=2).reshape(OHu * G, KH * width)
            y = jnp.dot(lhs, w_ref[...], preferred_element_type=jnp.float32)
            y = jnp.maximum(y + b_ref[...], 0.0)
            y3 = y.reshape(OHu, G, ncols * Cout)
            for s in range(ncols // 2):
                m = jnp.maximum(y3[:, :, (2 * s) * Cout:(2 * s + 1) * Cout],
                                y3[:, :, (2 * s + 1) * Cout:(2 * s + 2) * Cout])
                v = jnp.max(m.reshape(PH, 2, G, Cout), axis=1)
                pw = pw0 + s
                o_ref[:, :, pw * Cout:(pw + 1) * Cout] = v.astype(jnp.bfloat16)
    return body


def _stage12_fused(x2d, w1, b1, w2, b2):
    H, N, WC = x2d.shape
    Cin, Cout, KH, KW = 3, 128, 7, 7
    W = WC // Cin
    OH, OW = H - KH + 1, W - KW + 1
    PH, PW = OH // 2, OW // 2

    w1r = w1.reshape(KH, KW * Cin, Cout)

    def widen(ncols):
        cols = [jnp.pad(w1r, ((0, 0), (s * Cin, (ncols - 1 - s) * Cin), (0, 0)))
                for s in range(ncols)]
        return jnp.stack(cols, axis=2).reshape(KH * (KW + ncols - 1) * Cin,
                                               ncols * Cout)

    w1c6 = widen(6)
    w1c4 = widen(4)
    b6 = jnp.concatenate([b1] * 6, axis=1)
    b4 = jnp.concatenate([b1] * 4, axis=1)

    chunks = []
    pw = 0
    while pw < PW:
        n = 3 if PW - pw >= 3 else PW - pw
        chunks.append((pw, 2 * n))
        pw += n

    G = 16 if N % 16 == 0 else (8 if N % 8 == 0 else 1)
    s1body = _make_stage1_body(G, Cin, Cout, KH, KW, PH, PW, chunks)

    C2in, C2out, K2H, K2W = Cout, 256, 5, 5
    O2H, O2W = PH - K2H + 1, PW - K2W + 1
    P2H, P2W = O2H // 2, O2W // 2
    s2body = _make_stage2_body(G, C2in, C2out, K2H, K2W, P2H, P2W)

    def body(x_ref, wc6_ref, wc4_ref, b6_ref, b4_ref, w2_ref, b2_ref,
             o_ref, h1_ref):
        s1body(x_ref, wc6_ref, wc4_ref, b6_ref, b4_ref, h1_ref)
        s2body(h1_ref, w2_ref, b2_ref, o_ref)

    return pl.pallas_call(
        body,
        out_shape=jax.ShapeDtypeStruct((P2H, N, P2W * C2out), jnp.bfloat16),
        grid=(N // G,),
        in_specs=[
            pl.BlockSpec((H, G, WC), lambda n: (0, n, 0)),
            pl.BlockSpec(w1c6.shape, lambda n: (0, 0)),
            pl.BlockSpec(w1c4.shape, lambda n: (0, 0)),
            pl.BlockSpec(b6.shape, lambda n: (0, 0)),
            pl.BlockSpec(b4.shape, lambda n: (0, 0)),
            pl.BlockSpec(w2.shape, lambda n: (0, 0)),
            pl.BlockSpec(b2.shape, lambda n: (0, 0)),
        ],
        out_specs=pl.BlockSpec((P2H, G, P2W * C2out), lambda n: (0, n, 0)),
        scratch_shapes=[pltpu.VMEM((PH, G, PW * Cout), jnp.bfloat16)],
        compiler_params=pltpu.CompilerParams(dimension_semantics=("parallel",)),
    )(x2d, w1c6, w1c4, b6, b4, w2, b2)


def _make_stage2_body(G, Cin, Cout, KH, KW, PH, PW):
    OHu = 2 * PH
    KWC = KW * Cin

    def conv_col(h_ref, w_ref, b_ref, ow):
        pieces = [h_ref[i:i + OHu, :, ow * Cin:ow * Cin + KWC] for i in range(KH)]
        lhs = jnp.concatenate(pieces, axis=2).reshape(OHu * G, KH * KWC)
        acc = jnp.dot(lhs, w_ref[...], preferred_element_type=jnp.float32)
        return jnp.maximum(acc + b_ref[...], 0.0)

    def body(h_ref, w_ref, b_ref, o_ref):
        for pw in range(PW):
            cmax = jnp.maximum(conv_col(h_ref, w_ref, b_ref, 2 * pw),
                               conv_col(h_ref, w_ref, b_ref, 2 * pw + 1))
            pooled = jnp.max(cmax.reshape(PH, 2, G, Cout), axis=1)
            o_ref[:, :, pw * Cout:(pw + 1) * Cout] = pooled.astype(jnp.bfloat16)
    return body


def _make_linear_body(PH, FW):
    def body(f_ref, w_ref, b_ref, o_ref):
        y = b_ref[...]
        for ph in range(PH):
            wl = w_ref[ph * FW:(ph + 1) * FW, :].astype(jnp.bfloat16)
            y = y + jnp.dot(f_ref[ph], wl, preferred_element_type=jnp.float32)
        o_ref[...] = y
    return body


def _linear(feats, w_lin, b_lin):
    PH, N, FW = feats.shape
    out = w_lin.shape[1]
    G = N // 2 if N % 2 == 0 else N
    return pl.pallas_call(
        _make_linear_body(PH, FW),
        out_shape=jax.ShapeDtypeStruct((N, out), jnp.float32),
        grid=(N // G,),
        in_specs=[
            pl.BlockSpec((PH, G, FW), lambda n: (0, n, 0)),
            pl.BlockSpec(w_lin.shape, lambda n: (0, 0)),
            pl.BlockSpec(b_lin.shape, lambda n: (0, 0)),
        ],
        out_specs=pl.BlockSpec((G, out), lambda n: (n, 0)),
        compiler_params=pltpu.CompilerParams(dimension_semantics=("parallel",)),
    )(feats, w_lin, b_lin)


@jax.jit
def _forward(x, w1, b1, w2, b2, w_lin, b_lin):
    N, C, H, W = x.shape
    x2d = jnp.transpose(x, (2, 0, 3, 1)).reshape(H, N, W * C).astype(jnp.bfloat16)
    feats = _stage12_fused(x2d, w1, b1, w2, b2)
    return _linear(feats, w_lin, b_lin)


def kernel(x, w1, b1, w2, b2, w_lin, b_lin):
    return _forward(x, w1, b1, w2, b2, w_lin, b_lin)
